# Initial kernel scaffold; baseline (speedup 1.0000x reference)
import functools

import jax
import jax.numpy as jnp
from jax import lax
from jax.experimental import pallas as pl
from jax.experimental.pallas import tpu as pltpu
from jax.experimental.pallas import tpu_sc as plsc

NC = 2
NS = 16
NW = NC * NS
K = 80
ZR = 25


def _make_sc_segsum(n, e, d, with_counts):
    ept = e // NW
    nchunks = ept // K
    rpt = n // NS
    mesh = plsc.VectorSubcoreMesh(core_axis_name="c", subcore_axis_name="s")

    out_type = [jax.ShapeDtypeStruct((NC, n, d), jnp.float32)]
    scratch = [
        pltpu.VMEM((K,), jnp.int32),
        pltpu.VMEM((K,), jnp.int32),
        pltpu.VMEM((K, d), jnp.float32),
        pltpu.VMEM((ZR, d), jnp.float32),
        pltpu.VMEM_SHARED((n, d), jnp.float32),
        pltpu.SemaphoreType.DMA,
    ]
    if with_counts:
        out_type.append(jax.ShapeDtypeStruct((NC, n, 16), jnp.float32))
        scratch += [
            pltpu.VMEM((K, 16), jnp.float32),
            pltpu.VMEM((ZR, 16), jnp.float32),
            pltpu.VMEM_SHARED((n, 16), jnp.float32),
        ]

    def body(x_hbm, src_hbm, dst_hbm, *rest):
        if with_counts:
            (sum_out, cnt_out, srcv, dstv, rowsv, zbuf, acc_sh, sem,
             onesv, zbuf2, cnt_sh) = rest
        else:
            sum_out, srcv, dstv, rowsv, zbuf, acc_sh, sem = rest
        cid = lax.axis_index("c")
        sid = lax.axis_index("s")
        wid = sid * NC + cid

        zeros16 = jnp.zeros((16,), jnp.float32)
        ones16 = jnp.ones((16,), jnp.float32)

        def zfill(i, _):
            r = i // (d // 16)
            c = i % (d // 16)
            zbuf[r, pl.ds(c * 16, 16)] = zeros16
            return 0
        lax.fori_loop(0, ZR * (d // 16), zfill, 0)
        if with_counts:
            def zfill2(i, _):
                zbuf2[i, :] = zeros16
                return 0
            lax.fori_loop(0, ZR, zfill2, 0)

            def ofill(i, _):
                onesv[i, :] = ones16
                return 0
            lax.fori_loop(0, K, ofill, 0)

        def zcopy(i, _):
            off = sid * rpt + i * ZR
            pltpu.sync_copy(zbuf, acc_sh.at[pl.ds(off, ZR)])
            if with_counts:
                pltpu.sync_copy(zbuf2, cnt_sh.at[pl.ds(off, ZR)])
            return 0
        lax.fori_loop(0, rpt // ZR, zcopy, 0)
        plsc.subcore_barrier()

        def chunk(c, _):
            base = wid * ept + c * K
            pltpu.sync_copy(src_hbm.at[pl.ds(base, K)], srcv)
            pltpu.sync_copy(dst_hbm.at[pl.ds(base, K)], dstv)
            pltpu.async_copy(x_hbm.at[srcv], rowsv, sem).wait()
            pltpu.sync_copy(rowsv, acc_sh.at[dstv], add=True)
            if with_counts:
                pltpu.sync_copy(onesv, cnt_sh.at[dstv], add=True)
            return 0
        lax.fori_loop(0, nchunks, chunk, 0)
        plsc.subcore_barrier()

        off = sid * rpt
        pltpu.sync_copy(acc_sh.at[pl.ds(off, rpt)],
                        sum_out.at[cid, pl.ds(off, rpt)])
        if with_counts:
            pltpu.sync_copy(cnt_sh.at[pl.ds(off, rpt)],
                            cnt_out.at[cid, pl.ds(off, rpt)])

    return pl.kernel(body, out_type=out_type, mesh=mesh,
                     scratch_types=scratch)


def _dotT(a, w):
    return lax.dot_general(a, w, (((1,), (1,)), ((), ())),
                           preferred_element_type=jnp.float32)


def _tc1_body(x_ref, s_ref, c_ref, w1l_ref, b1l_ref, w1r_ref, w2l_ref,
              w2r_ref, b2l_ref, g_ref, pre2_ref):
    cnt = c_ref[0][:, 0:1] + c_ref[1][:, 0:1]
    inv = 1.0 / jnp.maximum(cnt, 1.0)
    agg = (s_ref[0] + s_ref[1]) * inv
    h = jnp.maximum(
        _dotT(agg, w1l_ref[...]) + b1l_ref[...] + _dotT(x_ref[...], w1r_ref[...]),
        0.0)
    g_ref[...] = _dotT(h, w2l_ref[...])
    pre2_ref[...] = _dotT(h, w2r_ref[...]) + b2l_ref[...]


def _tc2_body(s_ref, c_ref, pre2_ref, out_ref):
    cnt = c_ref[0][:, 0:1] + c_ref[1][:, 0:1]
    inv = 1.0 / jnp.maximum(cnt, 1.0)
    z = (s_ref[0] + s_ref[1]) * inv + pre2_ref[...]
    m = jnp.max(z, axis=1, keepdims=True)
    zs = z - m
    out_ref[...] = zs - jnp.log(jnp.sum(jnp.exp(zs), axis=1, keepdims=True))


def kernel(x, edge_index, W1l, b1l, W1r, W2l, b2l, W2r):
    n, d = x.shape
    e = edge_index.shape[1]
    h_dim = W1l.shape[0]
    c_dim = W2l.shape[0]
    assert e % (NW * K) == 0 and n % (NS * ZR) == 0

    src = edge_index[0]
    dst = edge_index[1]

    sc_a = _make_sc_segsum(n, e, d, with_counts=True)
    s1p, cntp = sc_a(x, src, dst)

    rb = n // 8
    grid = (n // rb,)
    full = lambda shape: pl.BlockSpec(shape, lambda i: (0,) * len(shape))
    rows = lambda m: pl.BlockSpec((rb, m), lambda i: (i, 0))
    parts = lambda m: pl.BlockSpec((NC, rb, m), lambda i: (0, i, 0))

    g, pre2 = pl.pallas_call(
        _tc1_body,
        grid=grid,
        in_specs=[rows(d), parts(d), parts(16), full((h_dim, d)),
                  full((1, h_dim)), full((h_dim, d)), full((c_dim, h_dim)),
                  full((c_dim, h_dim)), full((1, c_dim))],
        out_specs=[rows(c_dim), rows(c_dim)],
        out_shape=[jax.ShapeDtypeStruct((n, c_dim), jnp.float32),
                   jax.ShapeDtypeStruct((n, c_dim), jnp.float32)],
    )(x, s1p, cntp, W1l, b1l.reshape(1, -1), W1r, W2l, W2r,
      b2l.reshape(1, -1))

    sc_b = _make_sc_segsum(n, e, c_dim, with_counts=False)
    (s2p,) = sc_b(g, src, dst)

    out = pl.pallas_call(
        _tc2_body,
        grid=grid,
        in_specs=[parts(c_dim), parts(16), rows(c_dim)],
        out_specs=rows(c_dim),
        out_shape=jax.ShapeDtypeStruct((n, c_dim), jnp.float32),
    )(s2p, cntp, pre2)
    return out


# speedup vs baseline: 4.3897x; 4.3897x over previous
import jax
import jax.numpy as jnp
from jax import lax
from jax.experimental import pallas as pl
from jax.experimental.pallas import tpu as pltpu
from jax.experimental.pallas import tpu_sc as plsc

NC = 2
NS = 16
NW = NC * NS
K = 80
ZR = 32


def _make_sc_segsum(np_, e, d, gather):
    ept = e // NW
    nchunks = ept // K
    rpt = np_ // NS
    mesh = plsc.VectorSubcoreMesh(core_axis_name="c", subcore_axis_name="s")

    out_type = [jax.ShapeDtypeStruct((NC, np_, d), jnp.float32)]
    scratch = [
        pltpu.VMEM((K,), jnp.int32),
        pltpu.VMEM((K,), jnp.int32),
        pltpu.VMEM((K, d), jnp.float32),
        pltpu.VMEM((ZR, d), jnp.float32),
        pltpu.VMEM_SHARED((np_, d), jnp.float32),
        pltpu.SemaphoreType.DMA,
    ]

    def body(x_hbm, src_hbm, dst_hbm, sum_out, srcv, dstv, rowsv, zbuf,
             acc_sh, sem):
        cid = lax.axis_index("c")
        sid = lax.axis_index("s")
        wid = sid * NC + cid

        zeros16 = jnp.zeros((16,), jnp.float32)

        def zfill(i, _):
            zbuf[i // (d // 16), pl.ds((i % (d // 16)) * 16, 16)] = zeros16
            return 0
        lax.fori_loop(0, ZR * (d // 16), zfill, 0)

        if not gather:
            ones16 = jnp.ones((16,), jnp.float32)

            def ofill(i, _):
                rowsv[i // (d // 16), pl.ds((i % (d // 16)) * 16, 16)] = ones16
                return 0
            lax.fori_loop(0, K * (d // 16), ofill, 0)

        def zcopy(i, _):
            pltpu.sync_copy(zbuf, acc_sh.at[pl.ds(sid * rpt + i * ZR, ZR)])
            return 0
        lax.fori_loop(0, rpt // ZR, zcopy, 0)
        plsc.subcore_barrier()

        def chunk(c, _):
            base = wid * ept + c * K
            pltpu.sync_copy(dst_hbm.at[pl.ds(base, K)], dstv)
            if gather:
                pltpu.sync_copy(src_hbm.at[pl.ds(base, K)], srcv)
                pltpu.async_copy(x_hbm.at[srcv], rowsv, sem).wait()
            pltpu.sync_copy(rowsv, acc_sh.at[dstv], add=True)
            return 0
        lax.fori_loop(0, nchunks, chunk, 0)
        plsc.subcore_barrier()

        off = sid * rpt
        pltpu.sync_copy(acc_sh.at[pl.ds(off, rpt)],
                        sum_out.at[cid, pl.ds(off, rpt)])

    return pl.kernel(body, out_type=out_type, mesh=mesh,
                     scratch_types=scratch)


def _dotT(a, w):
    return lax.dot_general(a, w, (((1,), (1,)), ((), ())),
                           preferred_element_type=jnp.float32)


def _tc1_body(x_ref, s_ref, c_ref, w1l_ref, b1l_ref, w1r_ref,
              w2r_ref, b2l_ref, h_ref, pre2_ref):
    cnt = c_ref[0][:, 0:1] + c_ref[1][:, 0:1]
    inv = 1.0 / jnp.maximum(cnt, 1.0)
    agg = (s_ref[0] + s_ref[1]) * inv
    h = jnp.maximum(
        _dotT(agg, w1l_ref[...]) + b1l_ref[...] + _dotT(x_ref[...], w1r_ref[...]),
        0.0)
    h_ref[...] = h
    pre2_ref[...] = _dotT(h, w2r_ref[...]) + b2l_ref[...]


def _tc2_body(s_ref, c_ref, pre2_ref, w2l_ref, out_ref):
    cnt = c_ref[0][:, 0:1] + c_ref[1][:, 0:1]
    inv = 1.0 / jnp.maximum(cnt, 1.0)
    agg = (s_ref[0] + s_ref[1]) * inv
    z = _dotT(agg, w2l_ref[...]) + pre2_ref[...]
    m = jnp.max(z, axis=1, keepdims=True)
    zs = z - m
    out_ref[...] = zs - jnp.log(jnp.sum(jnp.exp(zs), axis=1, keepdims=True))


def kernel(x, edge_index, W1l, b1l, W1r, W2l, b2l, W2r):
    n, d = x.shape
    e = edge_index.shape[1]
    h_dim = W1l.shape[0]
    c_dim = W2l.shape[0]
    np_ = ((n + NS * ZR - 1) // (NS * ZR)) * (NS * ZR)
    assert e % (NW * K) == 0 and (np_ // NS) % 8 == 0

    src = edge_index[0]
    dst = edge_index[1]

    sc_cnt = _make_sc_segsum(np_, e, d, gather=False)
    (cntp,) = sc_cnt(x, src, dst)

    sc_a = _make_sc_segsum(np_, e, d, gather=True)
    (s1p,) = sc_a(x, src, dst)

    rb = 2000
    grid = (n // rb,)
    full = lambda shape: pl.BlockSpec(shape, lambda i: (0,) * len(shape))
    rows = lambda m: pl.BlockSpec((rb, m), lambda i: (i, 0))
    parts = lambda m: pl.BlockSpec((NC, rb, m), lambda i: (0, i, 0))

    h, pre2 = pl.pallas_call(
        _tc1_body,
        grid=grid,
        in_specs=[rows(d), parts(d), parts(d), full((h_dim, d)),
                  full((1, h_dim)), full((h_dim, d)),
                  full((c_dim, h_dim)), full((1, c_dim))],
        out_specs=[rows(h_dim), rows(c_dim)],
        out_shape=[jax.ShapeDtypeStruct((n, h_dim), jnp.float32),
                   jax.ShapeDtypeStruct((n, c_dim), jnp.float32)],
    )(x, s1p, cntp, W1l, b1l.reshape(1, -1), W1r, W2r,
      b2l.reshape(1, -1))

    sc_b = _make_sc_segsum(np_, e, h_dim, gather=True)
    (s2p,) = sc_b(h, src, dst)

    out = pl.pallas_call(
        _tc2_body,
        grid=grid,
        in_specs=[parts(h_dim), parts(d), rows(c_dim), full((c_dim, h_dim))],
        out_specs=rows(c_dim),
        out_shape=jax.ShapeDtypeStruct((n, c_dim), jnp.float32),
    )(s2p, cntp, pre2, W2l)
    return out


# speedup vs baseline: 9.1688x; 2.0887x over previous
import jax
import jax.numpy as jnp
from jax import lax
from jax.experimental import pallas as pl
from jax.experimental.pallas import tpu as pltpu
from jax.experimental.pallas import tpu_sc as plsc

NC = 2
NS = 16
NW = NC * NS
CK = 100
GP = 20
ZR = 16


def _make_sc_segsum(np_, e, d, gather):
    ept = e // NW
    nchunks = ept // CK
    ngroups = nchunks // GP
    assert nchunks % GP == 0 and GP % 2 == 0
    rpt = np_ // NS
    mesh = plsc.VectorSubcoreMesh(core_axis_name="c", subcore_axis_name="s")

    out_type = [jax.ShapeDtypeStruct((NC, np_, d), jnp.float32)]
    scratch = [
        pltpu.VMEM((GP, CK), jnp.int32),
        pltpu.VMEM((GP, CK), jnp.int32),
        pltpu.VMEM((CK, d), jnp.float32),
        pltpu.VMEM((CK, d), jnp.float32),
        pltpu.VMEM((ZR, d), jnp.float32),
        pltpu.VMEM_SHARED((np_, d), jnp.float32),
        pltpu.SemaphoreType.DMA,
        pltpu.SemaphoreType.DMA,
    ]

    def body(x_hbm, src_hbm, dst_hbm, sum_out, srcv, dstv, rows0, rows1,
             zbuf, acc_sh, sem0, sem1):
        cid = lax.axis_index("c")
        sid = lax.axis_index("s")
        wid = sid * NC + cid

        zeros16 = jnp.zeros((16,), jnp.float32)

        def zfill(i, _):
            zbuf[i // (d // 16), pl.ds((i % (d // 16)) * 16, 16)] = zeros16
            return 0
        lax.fori_loop(0, ZR * (d // 16), zfill, 0)

        if not gather:
            ones16 = jnp.ones((16,), jnp.float32)

            def ofill(i, _):
                rows0[i // (d // 16), pl.ds((i % (d // 16)) * 16, 16)] = ones16
                return 0
            lax.fori_loop(0, CK * (d // 16), ofill, 0)

        def zcopy(i, _):
            pltpu.sync_copy(zbuf, acc_sh.at[pl.ds(sid * rpt + i * ZR, ZR)])
            return 0
        lax.fori_loop(0, rpt // ZR, zcopy, 0)
        plsc.subcore_barrier()

        if gather:
            def group(g, _):
                pltpu.sync_copy(dst_hbm.at[wid, g], dstv)
                pltpu.sync_copy(src_hbm.at[wid, g], srcv)
                pltpu.async_copy(x_hbm.at[srcv.at[0]], rows0, sem0)

                def pair(i, _):
                    j0 = 2 * i
                    pltpu.async_copy(x_hbm.at[srcv.at[j0 + 1]], rows1, sem1)
                    pltpu.make_async_copy(
                        x_hbm.at[srcv.at[j0]], rows0, sem0).wait()
                    pltpu.sync_copy(rows0, acc_sh.at[dstv.at[j0]], add=True)

                    @pl.when(j0 + 2 < GP)
                    def _():
                        pltpu.async_copy(x_hbm.at[srcv.at[j0 + 2]], rows0, sem0)
                    pltpu.make_async_copy(
                        x_hbm.at[srcv.at[j0 + 1]], rows1, sem1).wait()
                    pltpu.sync_copy(rows1, acc_sh.at[dstv.at[j0 + 1]], add=True)
                    return 0
                lax.fori_loop(0, GP // 2, pair, 0)
                return 0
            lax.fori_loop(0, ngroups, group, 0)
        else:
            def group(g, _):
                pltpu.sync_copy(dst_hbm.at[wid, g], dstv)

                def chunk(j, _):
                    pltpu.sync_copy(rows0, acc_sh.at[dstv.at[j]], add=True)
                    return 0
                lax.fori_loop(0, GP, chunk, 0)
                return 0
            lax.fori_loop(0, ngroups, group, 0)
        plsc.subcore_barrier()

        off = sid * rpt
        pltpu.sync_copy(acc_sh.at[pl.ds(off, rpt)],
                        sum_out.at[cid, pl.ds(off, rpt)])

    return pl.kernel(body, out_type=out_type, mesh=mesh,
                     scratch_types=scratch)


def _dotT(a, w):
    return lax.dot_general(a, w, (((1,), (1,)), ((), ())),
                           preferred_element_type=jnp.float32)


def _tc1_body(x_ref, s_ref, c_ref, w1l_ref, b1l_ref, w1r_ref,
              w2r_ref, b2l_ref, h_ref, pre2_ref):
    cnt = c_ref[0][:, 0:1] + c_ref[1][:, 0:1]
    inv = 1.0 / jnp.maximum(cnt, 1.0)
    agg = (s_ref[0] + s_ref[1]) * inv
    h = jnp.maximum(
        _dotT(agg, w1l_ref[...]) + b1l_ref[...] + _dotT(x_ref[...], w1r_ref[...]),
        0.0)
    h_ref[...] = h
    pre2_ref[...] = _dotT(h, w2r_ref[...]) + b2l_ref[...]


def _tc2_body(s_ref, c_ref, pre2_ref, w2l_ref, out_ref):
    cnt = c_ref[0][:, 0:1] + c_ref[1][:, 0:1]
    inv = 1.0 / jnp.maximum(cnt, 1.0)
    agg = (s_ref[0] + s_ref[1]) * inv
    z = _dotT(agg, w2l_ref[...]) + pre2_ref[...]
    m = jnp.max(z, axis=1, keepdims=True)
    zs = z - m
    out_ref[...] = zs - jnp.log(jnp.sum(jnp.exp(zs), axis=1, keepdims=True))


def kernel(x, edge_index, W1l, b1l, W1r, W2l, b2l, W2r):
    n, d = x.shape
    e = edge_index.shape[1]
    h_dim = W1l.shape[0]
    c_dim = W2l.shape[0]
    np_ = ((n + NS * ZR - 1) // (NS * ZR)) * (NS * ZR)
    assert e % (NW * CK * GP) == 0 and (np_ // NS) % ZR == 0
    ngroups = e // (NW * CK * GP)

    src = edge_index[0].reshape(NW, ngroups, GP, CK)
    dst = edge_index[1].reshape(NW, ngroups, GP, CK)

    sc_cnt = _make_sc_segsum(np_, e, d, gather=False)
    (cntp,) = sc_cnt(x, src, dst)

    sc_a = _make_sc_segsum(np_, e, d, gather=True)
    (s1p,) = sc_a(x, src, dst)

    rb = 2000
    grid = (n // rb,)
    full = lambda shape: pl.BlockSpec(shape, lambda i: (0,) * len(shape))
    rows = lambda m: pl.BlockSpec((rb, m), lambda i: (i, 0))
    parts = lambda m: pl.BlockSpec((NC, rb, m), lambda i: (0, i, 0))

    h, pre2 = pl.pallas_call(
        _tc1_body,
        grid=grid,
        in_specs=[rows(d), parts(d), parts(d), full((h_dim, d)),
                  full((1, h_dim)), full((h_dim, d)),
                  full((c_dim, h_dim)), full((1, c_dim))],
        out_specs=[rows(h_dim), rows(c_dim)],
        out_shape=[jax.ShapeDtypeStruct((n, h_dim), jnp.float32),
                   jax.ShapeDtypeStruct((n, c_dim), jnp.float32)],
    )(x, s1p, cntp, W1l, b1l.reshape(1, -1), W1r, W2r,
      b2l.reshape(1, -1))

    sc_b = _make_sc_segsum(np_, e, h_dim, gather=True)
    (s2p,) = sc_b(h, src, dst)

    out = pl.pallas_call(
        _tc2_body,
        grid=grid,
        in_specs=[parts(h_dim), parts(d), rows(c_dim), full((c_dim, h_dim))],
        out_specs=rows(c_dim),
        out_shape=jax.ShapeDtypeStruct((n, c_dim), jnp.float32),
    )(s2p, cntp, pre2, W2l)
    return out
